# baseline (speedup 1.0000x reference)
import functools

import jax
import jax.numpy as jnp
from jax.experimental import pallas as pl
from jax.experimental.pallas import tpu as pltpu


def _round_up(x, m):
    return (x + m - 1) // m * m


def _finalize(psum_scr, psq_scr, gamma, beta, inv_n):
    total = jnp.sum(psum_scr[...], axis=0, keepdims=True) * 0.125
    total_sq = jnp.sum(psq_scr[...], axis=0, keepdims=True) * 0.125
    mean = total * inv_n
    var = jnp.maximum(total_sq * inv_n - mean * mean, 0.0)
    inv_std = jax.lax.rsqrt(var + 1e-5)
    scale = gamma * inv_std
    shift = beta - mean * scale
    return scale, shift


def _fused_body(x_ref, w1_ref, g1_ref, b1_ref, w2_ref, g2_ref, b2_ref,
                adj_ref, out_ref, acc_scr, xw_scr, psum_scr, psq_scr,
                *, tm, inv_n):
    i = pl.program_id(0)

    def dot(a, b):
        return jnp.dot(a, b, preferred_element_type=jnp.float32)

    def dot_ta(a, b):
        return jax.lax.dot_general(a, b, (((0,), (0,)), ((), ())),
                                   preferred_element_type=jnp.float32)

    def stats(h, first):
        ps = jnp.broadcast_to(jnp.sum(h, axis=0, keepdims=True),
                              psum_scr.shape)
        pq = jnp.broadcast_to(jnp.sum(h * h, axis=0, keepdims=True),
                              psq_scr.shape)
        if first:
            psum_scr[...] = ps
            psq_scr[...] = pq
        else:
            psum_scr[...] += ps
            psq_scr[...] += pq

    @pl.when(i == 0)
    def _():
        xw1 = dot(x_ref[...].astype(jnp.bfloat16),
                  w1_ref[...].astype(jnp.bfloat16)).astype(jnp.bfloat16)
        xw_scr[...] = xw1
        acc_scr[0:tm, :] = dot(adj_ref[...], xw1[0:tm, :])

    @pl.when(i == 1)
    def _():
        a = adj_ref[...]
        acc_scr[0:tm, :] += dot(a, xw_scr[tm:, :])
        acc_scr[tm:, :] = dot_ta(a, xw_scr[0:tm, :])
        stats(acc_scr[0:tm, :], first=True)

    @pl.when(i == 2)
    def _():
        acc_scr[tm:, :] += dot(adj_ref[...], xw_scr[tm:, :])
        stats(acc_scr[tm:, :], first=False)

    @pl.when(i == 3)
    def _():
        scale, shift = _finalize(psum_scr, psq_scr, g1_ref[...], b1_ref[...],
                                 inv_n)
        a1 = jnp.maximum(acc_scr[...] * scale + shift, 0.0)
        xw2 = dot(a1.astype(jnp.bfloat16),
                  w2_ref[...].astype(jnp.bfloat16)).astype(jnp.bfloat16)
        xw_scr[...] = xw2
        acc_scr[0:tm, :] = dot(adj_ref[...], xw2[0:tm, :])

    @pl.when(i == 4)
    def _():
        a = adj_ref[...]
        acc_scr[0:tm, :] += dot(a, xw_scr[tm:, :])
        acc_scr[tm:, :] = dot_ta(a, xw_scr[0:tm, :])
        stats(acc_scr[0:tm, :], first=True)

    @pl.when(i == 5)
    def _():
        acc_scr[tm:, :] += dot(adj_ref[...], xw_scr[tm:, :])
        stats(acc_scr[tm:, :], first=False)

    @pl.when(i == 6)
    def _():
        scale, shift = _finalize(psum_scr, psq_scr, g2_ref[...], b2_ref[...],
                                 inv_n)
        out_ref[...] = jnp.maximum(acc_scr[...] * scale + shift, 0.0)


def _adj_index(i):
    j = jnp.minimum(i - 3 * (i >= 3).astype(jnp.int32), 2)
    return (jnp.maximum(j - 1, 0), jnp.minimum(j, 1))


def _gcn_fused(x_pad, w1p, g1, b1, w2p, g2, b2, adj_pad, n_real):
    n_pad = adj_pad.shape[0]
    f_pad = w1p.shape[1]
    tm = n_pad // 2
    body = functools.partial(_fused_body, tm=tm, inv_n=1.0 / n_real)
    return pl.pallas_call(
        body,
        out_shape=jax.ShapeDtypeStruct((n_pad, f_pad), jnp.float32),
        grid=(7,),
        in_specs=[
            pl.BlockSpec(x_pad.shape, lambda i: (0, 0)),
            pl.BlockSpec(w1p.shape, lambda i: (0, 0)),
            pl.BlockSpec((1, f_pad), lambda i: (0, 0)),
            pl.BlockSpec((1, f_pad), lambda i: (0, 0)),
            pl.BlockSpec(w2p.shape, lambda i: (0, 0)),
            pl.BlockSpec((1, f_pad), lambda i: (0, 0)),
            pl.BlockSpec((1, f_pad), lambda i: (0, 0)),
            pl.BlockSpec((tm, tm), _adj_index),
        ],
        out_specs=pl.BlockSpec((n_pad, f_pad), lambda i: (0, 0)),
        scratch_shapes=[
            pltpu.VMEM((n_pad, f_pad), jnp.float32),
            pltpu.VMEM((n_pad, f_pad), jnp.bfloat16),
            pltpu.VMEM((8, f_pad), jnp.float32),
            pltpu.VMEM((8, f_pad), jnp.float32),
        ],
        compiler_params=pltpu.CompilerParams(
            dimension_semantics=("arbitrary",),
            vmem_limit_bytes=48 * 1024 * 1024),
    )(x_pad, w1p, g1, b1, w2p, g2, b2, adj_pad)


@functools.partial(jax.jit, static_argnames=("num_nodes",))
def _forward(w1, gamma1, beta1, w2, gamma2, beta2, x, adj_pad, num_nodes):
    n = num_nodes
    n_pad = adj_pad.shape[0]
    in_dim = x.shape[1]
    h_dim = w1.shape[1]
    out_dim = w2.shape[1]
    f1_pad = _round_up(h_dim, 128)
    f2_pad = _round_up(out_dim, 128)

    def pad_cols(v, f_pad):
        if v.shape[-1] == f_pad:
            return v.reshape(1, f_pad)
        return jnp.zeros((1, f_pad), jnp.float32).at[:, :v.shape[-1]].set(
            v.reshape(1, -1))

    x_pad = x
    if n_pad != n:
        x_pad = jnp.zeros((n_pad, in_dim), x.dtype).at[:n].set(x)

    w1p = w1
    if h_dim != f1_pad:
        w1p = jnp.zeros((in_dim, f1_pad), jnp.float32).at[:, :h_dim].set(w1)
    w2p = w2
    if h_dim != f1_pad or out_dim != f2_pad:
        w2p = jnp.zeros((f1_pad, f2_pad), jnp.float32)
        w2p = w2p.at[:h_dim, :out_dim].set(w2)

    if f1_pad != f2_pad:
        raise NotImplementedError("fused path expects equal padded widths")

    out = _gcn_fused(x_pad, w1p,
                     pad_cols(gamma1, f1_pad), pad_cols(beta1, f1_pad),
                     w2p, pad_cols(gamma2, f2_pad), pad_cols(beta2, f2_pad),
                     adj_pad, n)
    if n_pad != n or f2_pad != out_dim:
        out = out[:n, :out_dim]
    return out


def kernel(w1, b1, gamma1, beta1, w2, b2, gamma2, beta2, x, adj_pad):
    return _forward(w1, gamma1, beta1, w2, gamma2, beta2, x, adj_pad,
                    num_nodes=x.shape[0])
